# Initial kernel scaffold; baseline (speedup 1.0000x reference)
import jax
import jax.numpy as jnp
from jax.experimental import pallas as pl


def kernel(x, edge_index, edge_attr, Wl, bl, W1, b1, W2, b2):
    raise NotImplementedError("write your pallas kernel here")


# speedup vs baseline: 3.1644x; 3.1644x over previous
import functools

import jax
import jax.numpy as jnp
from jax import lax
from jax.experimental import pallas as pl
from jax.experimental.pallas import tpu as pltpu
from jax.experimental.pallas import tpu_sc as plsc

N_NODES = 10000
NIN = 128
NOUT = 128
N_EDGES = 320000

_NC = 2
_NS = 16
_NW = _NC * _NS
_EW = N_EDGES // _NW
_C = 80
_CH = _EW // _C


def _node_body(x_ref, wlt_ref, bl_ref, w1at_ref, y_ref):
    xl = jnp.dot(x_ref[...], wlt_ref[...], preferred_element_type=jnp.float32)
    xl = xl + bl_ref[...]
    y_ref[...] = jnp.dot(xl, w1at_ref[...], preferred_element_type=jnp.float32)


def _node_precompute(x, WlT, bl2, W1aT):
    return pl.pallas_call(
        _node_body,
        out_shape=jax.ShapeDtypeStruct((N_NODES, NIN), jnp.float32),
    )(x, WlT, bl2, W1aT)


def _gather_add_body(y_hbm, src_hbm, dst_hbm, g_hbm,
                     sidx, didx, srows, drows, sem_s, sem_d):
    wid = lax.axis_index("s") * _NC + lax.axis_index("c")
    pltpu.sync_copy(src_hbm.at[wid], sidx)
    pltpu.sync_copy(dst_hbm.at[wid], didx)

    def chunk(i, _):
        cs = pltpu.async_copy(y_hbm.at[sidx.at[i]], srows, sem_s)
        cd = pltpu.async_copy(y_hbm.at[didx.at[i]], drows, sem_d)
        cs.wait()
        cd.wait()

        def row(r, _):
            def col(j, _):
                k = j * 16
                srows[r, pl.ds(k, 16)] = (
                    srows[r, pl.ds(k, 16)] + drows[r, pl.ds(k, 16)])
                return 0
            return lax.fori_loop(0, NIN // 16, col, 0)

        lax.fori_loop(0, _C, row, 0)
        base = wid * _EW + i * _C
        pltpu.sync_copy(srows, g_hbm.at[pl.ds(base, _C)])
        return 0

    lax.fori_loop(0, _CH, chunk, 0)


def _gather_add(y, src3, dst3):
    mesh = plsc.VectorSubcoreMesh(core_axis_name="c", subcore_axis_name="s")
    fn = functools.partial(
        pl.kernel, mesh=mesh,
        out_type=jax.ShapeDtypeStruct((N_EDGES, NIN), jnp.float32),
        scratch_types=[
            pltpu.VMEM((_CH, _C), jnp.int32),
            pltpu.VMEM((_CH, _C), jnp.int32),
            pltpu.VMEM((_C, NIN), jnp.float32),
            pltpu.VMEM((_C, NIN), jnp.float32),
            pltpu.SemaphoreType.DMA,
            pltpu.SemaphoreType.DMA,
        ],
    )(_gather_add_body)
    return fn(y, src3, dst3)


_EB = 2560


def _edge_body(g_ref, ea_ref, w1et_ref, b1_ref, w2t_ref, b2_ref, out_ref):
    h = g_ref[...] + jnp.dot(ea_ref[...], w1et_ref[...],
                             preferred_element_type=jnp.float32) + b1_ref[...]
    h = jnp.maximum(h, 0.0)
    o = jnp.dot(h, w2t_ref[...], preferred_element_type=jnp.float32) + b2_ref[...]
    out_ref[...] = jnp.maximum(o, 0.0)


def _edge_mlp(g, edge_attr, W1eT, b12, W2T, b22):
    nblk = N_EDGES // _EB
    return pl.pallas_call(
        _edge_body,
        grid=(nblk,),
        in_specs=[
            pl.BlockSpec((_EB, NIN), lambda i: (i, 0)),
            pl.BlockSpec((_EB, NIN), lambda i: (i, 0)),
            pl.BlockSpec((NIN, NOUT), lambda i: (0, 0)),
            pl.BlockSpec((1, NOUT), lambda i: (0, 0)),
            pl.BlockSpec((NOUT, NOUT), lambda i: (0, 0)),
            pl.BlockSpec((1, NOUT), lambda i: (0, 0)),
        ],
        out_specs=pl.BlockSpec((_EB, NOUT), lambda i: (i, 0)),
        out_shape=jax.ShapeDtypeStruct((N_EDGES, NOUT), jnp.float32),
    )(g, edge_attr, W1eT, b12, W2T, b22)


def kernel(x, edge_index, edge_attr, Wl, bl, W1, b1, W2, b2):
    src3 = edge_index[0].astype(jnp.int32).reshape(_NW, _CH, _C)
    dst3 = edge_index[1].astype(jnp.int32).reshape(_NW, _CH, _C)
    WlT = Wl.T
    W1aT = W1[:, :NIN].T
    W1eT = W1[:, NIN:].T
    W2T = W2.T

    y = _node_precompute(x, WlT, bl.reshape(1, NIN), W1aT)
    g = _gather_add(y, src3, dst3)
    return _edge_mlp(g, edge_attr, W1eT, b1.reshape(1, NOUT), W2T,
                     b2.reshape(1, NOUT))
